# baseline (speedup 1.0000x reference)
import functools

import jax
import jax.numpy as jnp
from jax import lax
from jax.experimental import pallas as pl
from jax.experimental.pallas import tpu as pltpu
from jax.experimental.pallas import tpu_sc as plsc

EPS = 1e-5
_S1 = 1.0 / (1.0 + EPS) ** 0.5

_NC = 2
_NS = 16
_NW = _NC * _NS


def _leaky(v):
    return jnp.where(v >= 0, v, 0.2 * v)


def _front_body(x_ref, w1t_ref, b1_ref, g1_ref, be1_ref, wct_ref,
                h_ref, xt_ref):
    h = jnp.dot(x_ref[...], w1t_ref[...], preferred_element_type=jnp.float32)
    h = _leaky(h + b1_ref[...])
    h = h * (_S1 * g1_ref[...]) + be1_ref[...]
    h_ref[...] = h
    xt = jnp.dot(h, wct_ref[...], preferred_element_type=jnp.float32)
    xt_ref[:, :128] = xt
    r = xt.shape[0]
    lane = lax.broadcasted_iota(jnp.int32, (r, 16), 1)
    xt_ref[:, 128:144] = jnp.where(lane == 0, 1.0, 0.0)


def _tc_front(x2d, w1t, b1, g1, be1, wct, n, blk):
    grid = n // blk
    return pl.pallas_call(
        _front_body,
        grid=(grid,),
        in_specs=[
            pl.BlockSpec((blk, 128), lambda i: (i, 0)),
            pl.BlockSpec((128, 128), lambda i: (0, 0)),
            pl.BlockSpec((1, 128), lambda i: (0, 0)),
            pl.BlockSpec((1, 128), lambda i: (0, 0)),
            pl.BlockSpec((1, 128), lambda i: (0, 0)),
            pl.BlockSpec((128, 128), lambda i: (0, 0)),
        ],
        out_specs=[
            pl.BlockSpec((blk, 128), lambda i: (i, 0)),
            pl.BlockSpec((blk, 144), lambda i: (i, 0)),
        ],
        out_shape=[
            jax.ShapeDtypeStruct((n, 128), jnp.float32),
            jax.ShapeDtypeStruct((n, 144), jnp.float32),
        ],
    )(x2d, w1t, b1, g1, be1, wct)


def _sc_pass(table, gidx, sidx):
    n, w = table.shape
    nw, nch, k = gidx.shape
    per_sub = n // _NS
    zk = min(k, 64)
    full, rem = per_sub // zk, per_sub % zk

    @functools.partial(
        pl.kernel,
        out_type=jax.ShapeDtypeStruct((_NC, n, w), jnp.float32),
        mesh=plsc.VectorSubcoreMesh(core_axis_name="c", subcore_axis_name="s",
                                    num_cores=_NC, num_subcores=_NS),
        scratch_types=[
            pltpu.VMEM((nch, k), jnp.int32),
            pltpu.VMEM((nch, k), jnp.int32),
            pltpu.VMEM((2, k, w), jnp.float32),
            pltpu.VMEM_SHARED((n + 8, w), jnp.float32),
            pltpu.SemaphoreType.DMA,
        ],
        compiler_params=pltpu.CompilerParams(use_tc_tiling_on_sc=False),
    )
    def body(table_hbm, gidx_hbm, sidx_hbm, out_hbm,
             gidx_v, sidx_v, rows_v, acc_sh, sem):
        cid = lax.axis_index("c")
        sid = lax.axis_index("s")
        wid = cid * _NS + sid

        zero16 = jnp.zeros((16,), jnp.float32)
        for i in range(zk):
            for j in range(w // 16):
                rows_v[0, i, pl.ds(j * 16, 16)] = zero16
        base = sid * per_sub
        for t in range(full):
            pltpu.sync_copy(rows_v.at[0, pl.ds(0, zk)],
                            acc_sh.at[pl.ds(base + t * zk, zk)])
        if rem:
            pltpu.sync_copy(rows_v.at[0, pl.ds(0, rem)],
                            acc_sh.at[pl.ds(base + full * zk, rem)])
        plsc.subcore_barrier()

        pltpu.sync_copy(gidx_hbm.at[wid], gidx_v)
        pltpu.sync_copy(sidx_hbm.at[wid], sidx_v)

        def gather(c, buf):
            return pltpu.make_async_copy(
                table_hbm.at[gidx_v.at[c]], rows_v.at[buf], sem)

        gather(0, 0).start()

        def chunk(c, carry):
            buf = lax.rem(c, 2)

            @pl.when(c + 1 < nch)
            def _():
                gather(c + 1, 1 - buf).start()

            gather(c, buf).wait()
            pltpu.sync_copy(rows_v.at[buf], acc_sh.at[sidx_v.at[c]], add=True)
            return carry

        lax.fori_loop(0, nch, chunk, 0)
        plsc.subcore_barrier()

        pltpu.sync_copy(acc_sh.at[pl.ds(base, per_sub)],
                        out_hbm.at[cid, pl.ds(base, per_sub)])

    return body(table, gidx, sidx)


def _comb_body(p0_ref, p1_ref, o_ref):
    s = p0_ref[...] + p1_ref[...]
    cnt = s[:, 128:129]
    inv = jnp.where(cnt == 0, 0.0, 1.0 / jnp.where(cnt == 0, 1.0, cnt))
    o_ref[:, :128] = s[:, :128] * inv
    r = s.shape[0]
    lane = lax.broadcasted_iota(jnp.int32, (r, 16), 1)
    o_ref[:, 128:144] = jnp.where(lane == 0, 1.0, 0.0)


def _tc_comb(p0, p1, n, blk):
    return pl.pallas_call(
        _comb_body,
        grid=(n // blk,),
        in_specs=[
            pl.BlockSpec((blk, 144), lambda i: (i, 0)),
            pl.BlockSpec((blk, 144), lambda i: (i, 0)),
        ],
        out_specs=pl.BlockSpec((blk, 144), lambda i: (i, 0)),
        out_shape=jax.ShapeDtypeStruct((n, 144), jnp.float32),
    )(p0, p1)


def _back_body(src_ref, h_ref, q0_ref, q1_ref, bc_ref, g2_ref, be2_ref,
               w2t_ref, b2_ref, lw_ref, lb_ref, o_ref):
    q = q0_ref[...] + q1_ref[...]
    dv = q[:, 128:129]
    dinv = jnp.where(dv == 0, 0.0, 1.0 / jnp.where(dv == 0, 1.0, dv))
    hh = h_ref[...] + q[:, :128] * dinv + bc_ref[...]
    hh = hh * (_S1 * g2_ref[...]) + be2_ref[...]
    g = jnp.dot(hh, w2t_ref[...], preferred_element_type=jnp.float32)
    g = _leaky(g + b2_ref[...])
    o = src_ref[...] + g
    mu = jnp.mean(o, axis=1, keepdims=True)
    var = jnp.mean((o - mu) ** 2, axis=1, keepdims=True)
    o_ref[...] = (o - mu) / jnp.sqrt(var + EPS) * lw_ref[...] + lb_ref[...]


def _tc_back(src2d, h, q0, q1, bc, g2, be2, w2t, b2, lw, lb, n, blk):
    vec = pl.BlockSpec((1, 128), lambda i: (0, 0))
    return pl.pallas_call(
        _back_body,
        grid=(n // blk,),
        in_specs=[
            pl.BlockSpec((blk, 128), lambda i: (i, 0)),
            pl.BlockSpec((blk, 128), lambda i: (i, 0)),
            pl.BlockSpec((blk, 144), lambda i: (i, 0)),
            pl.BlockSpec((blk, 144), lambda i: (i, 0)),
            vec, vec, vec,
            pl.BlockSpec((128, 128), lambda i: (0, 0)),
            vec, vec, vec,
        ],
        out_specs=pl.BlockSpec((blk, 128), lambda i: (i, 0)),
        out_shape=jax.ShapeDtypeStruct((n, 128), jnp.float32),
    )(src2d, h, q0, q1, bc, g2, be2, w2t, b2, lw, lb)


def kernel(x, hyperedge_all, lin1_W, lin1_b, bn1_w, bn1_b, hconv_W, hconv_b,
           bn2_w, bn2_b, lin2_W, lin2_b, ln_w, ln_b):
    b_, n, c = x.shape
    nnz = hyperedge_all.shape[1]
    per_w = nnz // _NW
    k = 64
    nch = -(-per_w // k)
    pad = nch * k - per_w

    x2d = x.reshape(n, b_ * c)

    def _pad_idx(idx, fill):
        idx2 = idx.reshape(_NW, per_w)
        idx2 = jnp.pad(idx2, ((0, 0), (0, pad)), constant_values=fill)
        return idx2.reshape(_NW, nch, k)

    row_g = _pad_idx(hyperedge_all[0], 0)
    row_s = _pad_idx(hyperedge_all[0], n)
    col_g = _pad_idx(hyperedge_all[1], 0)
    col_s = _pad_idx(hyperedge_all[1], n)

    blk = 1000
    h, xt_ext = _tc_front(
        x2d, lin1_W.T, lin1_b.reshape(1, -1), bn1_w.reshape(1, -1),
        bn1_b.reshape(1, -1), hconv_W.T, n, blk)

    p = _sc_pass(xt_ext, row_g, col_s)
    out_e = _tc_comb(p[0], p[1], n, blk)
    q = _sc_pass(out_e, col_g, row_s)

    out2d = _tc_back(
        x2d, h, q[0], q[1], hconv_b.reshape(1, -1), bn2_w.reshape(1, -1),
        bn2_b.reshape(1, -1), lin2_W.T, lin2_b.reshape(1, -1),
        ln_w.reshape(1, -1), ln_b.reshape(1, -1), n, blk)
    return out2d.reshape(b_, n, c)


# speedup vs baseline: 1.0599x; 1.0599x over previous
import functools

import jax
import jax.numpy as jnp
from jax import lax
from jax.experimental import pallas as pl
from jax.experimental.pallas import tpu as pltpu
from jax.experimental.pallas import tpu_sc as plsc

EPS = 1e-5
_S1 = 1.0 / (1.0 + EPS) ** 0.5

_NC = 2
_NS = 16
_NW = _NC * _NS


def _leaky(v):
    return jnp.where(v >= 0, v, 0.2 * v)


def _front_body(x_ref, w1t_ref, b1_ref, g1_ref, be1_ref, wct_ref,
                h_ref, xt_ref):
    h = jnp.dot(x_ref[...], w1t_ref[...], preferred_element_type=jnp.float32)
    h = _leaky(h + b1_ref[...])
    h = h * (_S1 * g1_ref[...]) + be1_ref[...]
    h_ref[...] = h
    xt = jnp.dot(h, wct_ref[...], preferred_element_type=jnp.float32)
    xt_ref[:, :128] = xt
    r = xt.shape[0]
    lane = lax.broadcasted_iota(jnp.int32, (r, 16), 1)
    xt_ref[:, 128:144] = jnp.where(lane == 0, 1.0, 0.0)


def _tc_front(x2d, w1t, b1, g1, be1, wct, n, blk):
    grid = n // blk
    return pl.pallas_call(
        _front_body,
        grid=(grid,),
        in_specs=[
            pl.BlockSpec((blk, 128), lambda i: (i, 0)),
            pl.BlockSpec((128, 128), lambda i: (0, 0)),
            pl.BlockSpec((1, 128), lambda i: (0, 0)),
            pl.BlockSpec((1, 128), lambda i: (0, 0)),
            pl.BlockSpec((1, 128), lambda i: (0, 0)),
            pl.BlockSpec((128, 128), lambda i: (0, 0)),
        ],
        out_specs=[
            pl.BlockSpec((blk, 128), lambda i: (i, 0)),
            pl.BlockSpec((blk, 144), lambda i: (i, 0)),
        ],
        out_shape=[
            jax.ShapeDtypeStruct((n, 128), jnp.float32),
            jax.ShapeDtypeStruct((n, 144), jnp.float32),
        ],
    )(x2d, w1t, b1, g1, be1, wct)


def _sc_pass(table, gidx, sidx):
    n, w = table.shape
    nw, nch, k = gidx.shape
    per_sub = n // _NS
    zk = min(k, 64)
    full, rem = per_sub // zk, per_sub % zk

    @functools.partial(
        pl.kernel,
        out_type=jax.ShapeDtypeStruct((_NC, n, w), jnp.float32),
        mesh=plsc.VectorSubcoreMesh(core_axis_name="c", subcore_axis_name="s",
                                    num_cores=_NC, num_subcores=_NS),
        scratch_types=[
            pltpu.VMEM((nch, k), jnp.int32),
            pltpu.VMEM((nch, k), jnp.int32),
            pltpu.VMEM((2, k, w), jnp.float32),
            pltpu.VMEM_SHARED((n + 8, w), jnp.float32),
            pltpu.SemaphoreType.DMA,
        ],
        compiler_params=pltpu.CompilerParams(use_tc_tiling_on_sc=False),
    )
    def body(table_hbm, gidx_hbm, sidx_hbm, out_hbm,
             gidx_v, sidx_v, rows_v, acc_sh, sem):
        cid = lax.axis_index("c")
        sid = lax.axis_index("s")
        wid = cid * _NS + sid

        zero16 = jnp.zeros((16,), jnp.float32)
        for i in range(zk):
            for j in range(w // 16):
                rows_v[0, i, pl.ds(j * 16, 16)] = zero16
        base = sid * per_sub
        for t in range(full):
            pltpu.sync_copy(rows_v.at[0, pl.ds(0, zk)],
                            acc_sh.at[pl.ds(base + t * zk, zk)])
        if rem:
            pltpu.sync_copy(rows_v.at[0, pl.ds(0, rem)],
                            acc_sh.at[pl.ds(base + full * zk, rem)])
        plsc.subcore_barrier()

        pltpu.sync_copy(gidx_hbm.at[wid], gidx_v)
        pltpu.sync_copy(sidx_hbm.at[wid], sidx_v)

        def gather(c, buf):
            return pltpu.make_async_copy(
                table_hbm.at[gidx_v.at[c]], rows_v.at[buf], sem)

        gather(0, 0).start()

        def chunk(c, carry):
            buf = lax.rem(c, 2)

            @pl.when(c + 1 < nch)
            def _():
                gather(c + 1, 1 - buf).start()

            gather(c, buf).wait()
            pltpu.sync_copy(rows_v.at[buf], acc_sh.at[sidx_v.at[c]], add=True)
            return carry

        lax.fori_loop(0, nch, chunk, 0)
        plsc.subcore_barrier()

        pltpu.sync_copy(acc_sh.at[pl.ds(base, per_sub)],
                        out_hbm.at[cid, pl.ds(base, per_sub)])

    return body(table, gidx, sidx)


def _comb_body(p0_ref, p1_ref, o_ref):
    s = p0_ref[...] + p1_ref[...]
    cnt = s[:, 128:129]
    inv = jnp.where(cnt == 0, 0.0, 1.0 / jnp.where(cnt == 0, 1.0, cnt))
    o_ref[:, :128] = s[:, :128] * inv
    r = s.shape[0]
    lane = lax.broadcasted_iota(jnp.int32, (r, 16), 1)
    o_ref[:, 128:144] = jnp.where(lane == 0, 1.0, 0.0)


def _tc_comb(p0, p1, n, blk):
    return pl.pallas_call(
        _comb_body,
        grid=(n // blk,),
        in_specs=[
            pl.BlockSpec((blk, 144), lambda i: (i, 0)),
            pl.BlockSpec((blk, 144), lambda i: (i, 0)),
        ],
        out_specs=pl.BlockSpec((blk, 144), lambda i: (i, 0)),
        out_shape=jax.ShapeDtypeStruct((n, 144), jnp.float32),
    )(p0, p1)


def _back_body(src_ref, h_ref, q0_ref, q1_ref, bc_ref, g2_ref, be2_ref,
               w2t_ref, b2_ref, lw_ref, lb_ref, o_ref):
    q = q0_ref[...] + q1_ref[...]
    dv = q[:, 128:129]
    dinv = jnp.where(dv == 0, 0.0, 1.0 / jnp.where(dv == 0, 1.0, dv))
    hh = h_ref[...] + q[:, :128] * dinv + bc_ref[...]
    hh = hh * (_S1 * g2_ref[...]) + be2_ref[...]
    g = jnp.dot(hh, w2t_ref[...], preferred_element_type=jnp.float32)
    g = _leaky(g + b2_ref[...])
    o = src_ref[...] + g
    mu = jnp.mean(o, axis=1, keepdims=True)
    var = jnp.mean((o - mu) ** 2, axis=1, keepdims=True)
    o_ref[...] = (o - mu) / jnp.sqrt(var + EPS) * lw_ref[...] + lb_ref[...]


def _tc_back(src2d, h, q0, q1, bc, g2, be2, w2t, b2, lw, lb, n, blk):
    vec = pl.BlockSpec((1, 128), lambda i: (0, 0))
    return pl.pallas_call(
        _back_body,
        grid=(n // blk,),
        in_specs=[
            pl.BlockSpec((blk, 128), lambda i: (i, 0)),
            pl.BlockSpec((blk, 128), lambda i: (i, 0)),
            pl.BlockSpec((blk, 144), lambda i: (i, 0)),
            pl.BlockSpec((blk, 144), lambda i: (i, 0)),
            vec, vec, vec,
            pl.BlockSpec((128, 128), lambda i: (0, 0)),
            vec, vec, vec,
        ],
        out_specs=pl.BlockSpec((blk, 128), lambda i: (i, 0)),
        out_shape=jax.ShapeDtypeStruct((n, 128), jnp.float32),
    )(src2d, h, q0, q1, bc, g2, be2, w2t, b2, lw, lb)


def kernel(x, hyperedge_all, lin1_W, lin1_b, bn1_w, bn1_b, hconv_W, hconv_b,
           bn2_w, bn2_b, lin2_W, lin2_b, ln_w, ln_b):
    b_, n, c = x.shape
    nnz = hyperedge_all.shape[1]
    per_w = nnz // _NW
    k = 32
    nch = -(-per_w // k)
    pad = nch * k - per_w

    x2d = x.reshape(n, b_ * c)

    def _pad_idx(idx, fill):
        idx2 = idx.reshape(_NW, per_w)
        idx2 = jnp.pad(idx2, ((0, 0), (0, pad)), constant_values=fill)
        return idx2.reshape(_NW, nch, k)

    row_g = _pad_idx(hyperedge_all[0], 0)
    row_s = _pad_idx(hyperedge_all[0], n)
    col_g = _pad_idx(hyperedge_all[1], 0)
    col_s = _pad_idx(hyperedge_all[1], n)

    blk = 1000
    h, xt_ext = _tc_front(
        x2d, lin1_W.T, lin1_b.reshape(1, -1), bn1_w.reshape(1, -1),
        bn1_b.reshape(1, -1), hconv_W.T, n, blk)

    p = _sc_pass(xt_ext, row_g, col_s)
    out_e = _tc_comb(p[0], p[1], n, blk)
    q = _sc_pass(out_e, col_g, row_s)

    out2d = _tc_back(
        x2d, h, q[0], q[1], hconv_b.reshape(1, -1), bn2_w.reshape(1, -1),
        bn2_b.reshape(1, -1), lin2_W.T, lin2_b.reshape(1, -1),
        ln_w.reshape(1, -1), ln_b.reshape(1, -1), n, blk)
    return out2d.reshape(b_, n, c)


# speedup vs baseline: 1.5822x; 1.4928x over previous
import functools

import jax
import jax.numpy as jnp
from jax import lax
from jax.experimental import pallas as pl
from jax.experimental.pallas import tpu as pltpu
from jax.experimental.pallas import tpu_sc as plsc

EPS = 1e-5
_S1 = 1.0 / (1.0 + EPS) ** 0.5

_NC = 2
_NS = 16
_HW = 64
_CB = 80


def _leaky(v):
    return jnp.where(v >= 0, v, 0.2 * v)


def _front_body(x_ref, w1t_ref, b1_ref, g1_ref, be1_ref, wct_ref,
                h_ref, ta_ref, tb_ref):
    h = jnp.dot(x_ref[...], w1t_ref[...], preferred_element_type=jnp.float32)
    h = _leaky(h + b1_ref[...])
    h = h * (_S1 * g1_ref[...]) + be1_ref[...]
    h_ref[...] = h
    xt = jnp.dot(h, wct_ref[...], preferred_element_type=jnp.float32)
    ta_ref[...] = xt[:, :_HW]
    tb_ref[...] = xt[:, _HW:]


def _tc_front(x2d, w1t, b1, g1, be1, wct, n, blk):
    return pl.pallas_call(
        _front_body,
        grid=(n // blk,),
        in_specs=[
            pl.BlockSpec((blk, 128), lambda i: (i, 0)),
            pl.BlockSpec((128, 128), lambda i: (0, 0)),
            pl.BlockSpec((1, 128), lambda i: (0, 0)),
            pl.BlockSpec((1, 128), lambda i: (0, 0)),
            pl.BlockSpec((1, 128), lambda i: (0, 0)),
            pl.BlockSpec((128, 128), lambda i: (0, 0)),
        ],
        out_specs=[
            pl.BlockSpec((blk, 128), lambda i: (i, 0)),
            pl.BlockSpec((blk, _HW), lambda i: (i, 0)),
            pl.BlockSpec((blk, _HW), lambda i: (i, 0)),
        ],
        out_shape=[
            jax.ShapeDtypeStruct((n, 128), jnp.float32),
            jax.ShapeDtypeStruct((n, _HW), jnp.float32),
            jax.ShapeDtypeStruct((n, _HW), jnp.float32),
        ],
    )(x2d, w1t, b1, g1, be1, wct)


def _sc_mega(ta, tb, ridx, cidx):
    n = ta.shape[0]
    ns, nch, k = ridx.shape
    per_sub = n // _NS
    nu = n // _CB
    nt = -(-nu // _NS)

    @functools.partial(
        pl.kernel,
        out_type=[
            jax.ShapeDtypeStruct((n, _HW), jnp.float32),
            jax.ShapeDtypeStruct((n, _HW), jnp.float32),
            jax.ShapeDtypeStruct((n,), jnp.float32),
        ],
        mesh=plsc.VectorSubcoreMesh(core_axis_name="c", subcore_axis_name="s",
                                    num_cores=_NC, num_subcores=_NS),
        scratch_types=[
            pltpu.VMEM((nch, k), jnp.int32),
            pltpu.VMEM((nch, k), jnp.int32),
            pltpu.VMEM((2, k, _HW), jnp.float32),
            pltpu.VMEM((_CB, _HW), jnp.float32),
            pltpu.VMEM((_CB,), jnp.float32),
            pltpu.VMEM((k,), jnp.float32),
            pltpu.VMEM_SHARED((n, _HW), jnp.float32),
            pltpu.VMEM_SHARED((n, _HW), jnp.float32),
            pltpu.VMEM_SHARED((n,), jnp.float32),
            pltpu.VMEM_SHARED((n,), jnp.float32),
            pltpu.SemaphoreType.DMA,
        ],
        compiler_params=pltpu.CompilerParams(use_tc_tiling_on_sc=False),
    )
    def body(ta_hbm, tb_hbm, ridx_hbm, cidx_hbm, qa_hbm, qb_hbm, dv_hbm,
             rowv, colv, rows_v, blk_v, cnt_v, ones_v,
             acc1, acc2, hist_c, hist_r, sem):
        cid = lax.axis_index("c")
        sid = lax.axis_index("s")

        zero16 = jnp.zeros((16,), jnp.float32)
        one16 = jnp.ones((16,), jnp.float32)
        for i in range(_CB):
            for j in range(_HW // 16):
                blk_v[i, pl.ds(j * 16, 16)] = zero16
        for j in range(_CB // 16):
            cnt_v[pl.ds(j * 16, 16)] = zero16
        for off in (0, 16, k - 16):
            ones_v[pl.ds(off, 16)] = one16

        def zunit(t, carry):
            u = sid + t * _NS

            @pl.when(u < nu)
            def _():
                r0 = u * _CB
                pltpu.sync_copy(blk_v, acc1.at[pl.ds(r0, _CB)])
                pltpu.sync_copy(blk_v, acc2.at[pl.ds(r0, _CB)])
                pltpu.sync_copy(cnt_v, hist_c.at[pl.ds(r0, _CB)])
                pltpu.sync_copy(cnt_v, hist_r.at[pl.ds(r0, _CB)])
            return carry

        lax.fori_loop(0, nt, zunit, 0)

        pltpu.sync_copy(ridx_hbm.at[sid], rowv)
        pltpu.sync_copy(cidx_hbm.at[sid], colv)
        plsc.subcore_barrier()

        def g1(c, buf):
            @pl.when(cid == 0)
            def _():
                pltpu.make_async_copy(
                    ta_hbm.at[rowv.at[c]], rows_v.at[buf], sem).start()

            @pl.when(cid == 1)
            def _():
                pltpu.make_async_copy(
                    tb_hbm.at[rowv.at[c]], rows_v.at[buf], sem).start()

        g1(0, 0)

        def chunk1(c, carry):
            buf = lax.rem(c, 2)

            @pl.when(c + 1 < nch)
            def _():
                g1(c + 1, 1 - buf)

            pltpu.make_async_copy(
                ta_hbm.at[rowv.at[c]], rows_v.at[buf], sem).wait()
            pltpu.sync_copy(rows_v.at[buf], acc1.at[colv.at[c]], add=True)
            pltpu.sync_copy(ones_v, hist_c.at[colv.at[c]], add=True)
            pltpu.sync_copy(ones_v, hist_r.at[rowv.at[c]], add=True)
            return carry

        lax.fori_loop(0, nch, chunk1, 0)
        plsc.subcore_barrier()

        def cunit(t, carry):
            u = sid + t * _NS

            @pl.when(u < nu)
            def _():
                r0 = u * _CB
                pltpu.sync_copy(acc1.at[pl.ds(r0, _CB)], blk_v)
                pltpu.sync_copy(hist_c.at[pl.ds(r0, _CB)], cnt_v)
                for g in range(_CB // 16):
                    cnt = cnt_v[pl.ds(g * 16, 16)]
                    inv = jnp.where(cnt == 0, 0.0,
                                    1.0 / jnp.where(cnt == 0, 1.0, cnt))
                    for j in range(16):
                        r = g * 16 + j
                        s = inv[j]
                        for seg in range(_HW // 16):
                            v = blk_v[r, pl.ds(seg * 16, 16)]
                            blk_v[r, pl.ds(seg * 16, 16)] = v * s
                pltpu.sync_copy(blk_v, acc1.at[pl.ds(r0, _CB)])
            return carry

        lax.fori_loop(0, nt, cunit, 0)
        plsc.subcore_barrier()

        def g2(c, buf):
            pltpu.make_async_copy(
                acc1.at[colv.at[c]], rows_v.at[buf], sem).start()

        g2(0, 0)

        def chunk2(c, carry):
            buf = lax.rem(c, 2)

            @pl.when(c + 1 < nch)
            def _():
                g2(c + 1, 1 - buf)

            pltpu.make_async_copy(
                acc1.at[colv.at[c]], rows_v.at[buf], sem).wait()
            pltpu.sync_copy(rows_v.at[buf], acc2.at[rowv.at[c]], add=True)
            return carry

        lax.fori_loop(0, nch, chunk2, 0)
        plsc.subcore_barrier()

        base = sid * per_sub

        @pl.when(cid == 0)
        def _():
            pltpu.sync_copy(acc2.at[pl.ds(base, per_sub)],
                            qa_hbm.at[pl.ds(base, per_sub)])

        @pl.when(cid == 1)
        def _():
            pltpu.sync_copy(acc2.at[pl.ds(base, per_sub)],
                            qb_hbm.at[pl.ds(base, per_sub)])

        @pl.when(jnp.logical_and(cid == 0, sid == 0))
        def _():
            pltpu.sync_copy(hist_r, dv_hbm)

    return body(ta, tb, ridx, cidx)


def _back_body(src_ref, h_ref, qa_ref, qb_ref, dv_ref, bc_ref, g2_ref,
               be2_ref, w2t_ref, b2_ref, lw_ref, lb_ref, o_ref):
    dv = dv_ref[...]
    dinv = jnp.where(dv == 0, 0.0, 1.0 / jnp.where(dv == 0, 1.0, dv))
    outv = jnp.concatenate([qa_ref[...], qb_ref[...]], axis=1) * dinv
    hh = h_ref[...] + outv + bc_ref[...]
    hh = hh * (_S1 * g2_ref[...]) + be2_ref[...]
    g = jnp.dot(hh, w2t_ref[...], preferred_element_type=jnp.float32)
    g = _leaky(g + b2_ref[...])
    o = src_ref[...] + g
    mu = jnp.mean(o, axis=1, keepdims=True)
    var = jnp.mean((o - mu) ** 2, axis=1, keepdims=True)
    o_ref[...] = (o - mu) / jnp.sqrt(var + EPS) * lw_ref[...] + lb_ref[...]


def _tc_back(src2d, h, qa, qb, dv2d, bc, g2, be2, w2t, b2, lw, lb, n, blk):
    vec = pl.BlockSpec((1, 128), lambda i: (0, 0))
    return pl.pallas_call(
        _back_body,
        grid=(n // blk,),
        in_specs=[
            pl.BlockSpec((blk, 128), lambda i: (i, 0)),
            pl.BlockSpec((blk, 128), lambda i: (i, 0)),
            pl.BlockSpec((blk, _HW), lambda i: (i, 0)),
            pl.BlockSpec((blk, _HW), lambda i: (i, 0)),
            pl.BlockSpec((blk, 1), lambda i: (i, 0)),
            vec, vec, vec,
            pl.BlockSpec((128, 128), lambda i: (0, 0)),
            vec, vec, vec,
        ],
        out_specs=pl.BlockSpec((blk, 128), lambda i: (i, 0)),
        out_shape=jax.ShapeDtypeStruct((n, 128), jnp.float32),
    )(src2d, h, qa, qb, dv2d, bc, g2, be2, w2t, b2, lw, lb)


def kernel(x, hyperedge_all, lin1_W, lin1_b, bn1_w, bn1_b, hconv_W, hconv_b,
           bn2_w, bn2_b, lin2_W, lin2_b, ln_w, ln_b):
    b_, n, c = x.shape
    nnz = hyperedge_all.shape[1]
    per_sub_e = nnz // _NS
    k = 40
    nch = per_sub_e // k

    x2d = x.reshape(n, b_ * c)
    ridx = hyperedge_all[0].reshape(_NS, nch, k)
    cidx = hyperedge_all[1].reshape(_NS, nch, k)

    blk = 1000
    h, ta, tb = _tc_front(
        x2d, lin1_W.T, lin1_b.reshape(1, -1), bn1_w.reshape(1, -1),
        bn1_b.reshape(1, -1), hconv_W.T, n, blk)

    qa, qb, dv = _sc_mega(ta, tb, ridx, cidx)

    out2d = _tc_back(
        x2d, h, qa, qb, dv.reshape(n, 1), hconv_b.reshape(1, -1),
        bn2_w.reshape(1, -1), bn2_b.reshape(1, -1), lin2_W.T,
        lin2_b.reshape(1, -1), ln_w.reshape(1, -1), ln_b.reshape(1, -1),
        n, blk)
    return out2d.reshape(b_, n, c)


# speedup vs baseline: 1.8863x; 1.1922x over previous
import functools

import jax
import jax.numpy as jnp
from jax import lax
from jax.experimental import pallas as pl
from jax.experimental.pallas import tpu as pltpu
from jax.experimental.pallas import tpu_sc as plsc

EPS = 1e-5
_S1 = 1.0 / (1.0 + EPS) ** 0.5

_NC = 2
_NS = 16
_HW = 64
_CB = 80


def _leaky(v):
    return jnp.where(v >= 0, v, 0.2 * v)


def _front_body(x_ref, w1t_ref, b1_ref, g1_ref, be1_ref, wct_ref,
                h_ref, ta_ref, tb_ref):
    h = jnp.dot(x_ref[...], w1t_ref[...], preferred_element_type=jnp.float32)
    h = _leaky(h + b1_ref[...])
    h = h * (_S1 * g1_ref[...]) + be1_ref[...]
    h_ref[...] = h
    xt = jnp.dot(h, wct_ref[...], preferred_element_type=jnp.float32)
    ta_ref[...] = xt[:, :_HW]
    tb_ref[...] = xt[:, _HW:]


def _tc_front(x2d, w1t, b1, g1, be1, wct, n, blk):
    return pl.pallas_call(
        _front_body,
        grid=(n // blk,),
        in_specs=[
            pl.BlockSpec((blk, 128), lambda i: (i, 0)),
            pl.BlockSpec((128, 128), lambda i: (0, 0)),
            pl.BlockSpec((1, 128), lambda i: (0, 0)),
            pl.BlockSpec((1, 128), lambda i: (0, 0)),
            pl.BlockSpec((1, 128), lambda i: (0, 0)),
            pl.BlockSpec((128, 128), lambda i: (0, 0)),
        ],
        out_specs=[
            pl.BlockSpec((blk, 128), lambda i: (i, 0)),
            pl.BlockSpec((blk, _HW), lambda i: (i, 0)),
            pl.BlockSpec((blk, _HW), lambda i: (i, 0)),
        ],
        out_shape=[
            jax.ShapeDtypeStruct((n, 128), jnp.float32),
            jax.ShapeDtypeStruct((n, _HW), jnp.float32),
            jax.ShapeDtypeStruct((n, _HW), jnp.float32),
        ],
    )(x2d, w1t, b1, g1, be1, wct)


def _sc_mega(ta, tb, ridx, cidx):
    n = ta.shape[0]
    ns, nch, k = ridx.shape
    per_sub = n // _NS
    nu = n // _CB
    nt = -(-nu // _NS)

    @functools.partial(
        pl.kernel,
        out_type=[
            jax.ShapeDtypeStruct((n, _HW), jnp.float32),
            jax.ShapeDtypeStruct((n, _HW), jnp.float32),
            jax.ShapeDtypeStruct((n,), jnp.float32),
        ],
        mesh=plsc.VectorSubcoreMesh(core_axis_name="c", subcore_axis_name="s",
                                    num_cores=_NC, num_subcores=_NS),
        scratch_types=[
            pltpu.VMEM((nch, k), jnp.int32),
            pltpu.VMEM((nch, k), jnp.int32),
            pltpu.VMEM((2, k, _HW), jnp.float32),
            pltpu.VMEM((_CB, _HW), jnp.float32),
            pltpu.VMEM((_CB,), jnp.float32),
            pltpu.VMEM((k,), jnp.float32),
            pltpu.VMEM_SHARED((n, _HW), jnp.float32),
            pltpu.VMEM_SHARED((n, _HW), jnp.float32),
            pltpu.VMEM_SHARED((n,), jnp.float32),
            pltpu.VMEM_SHARED((n,), jnp.float32),
            pltpu.SemaphoreType.DMA,
        ],
        compiler_params=pltpu.CompilerParams(use_tc_tiling_on_sc=False),
    )
    def body(ta_hbm, tb_hbm, ridx_hbm, cidx_hbm, qa_hbm, qb_hbm, dv_hbm,
             rowv, colv, rows_v, blk_v, cnt_v, ones_v,
             acc1, acc2, hist_c, hist_r, sem):
        cid = lax.axis_index("c")
        sid = lax.axis_index("s")

        zero16 = jnp.zeros((16,), jnp.float32)
        one16 = jnp.ones((16,), jnp.float32)
        for i in range(_CB):
            for j in range(_HW // 16):
                blk_v[i, pl.ds(j * 16, 16)] = zero16
        for j in range(_CB // 16):
            cnt_v[pl.ds(j * 16, 16)] = zero16
        for off in (0, 16, k - 16):
            ones_v[pl.ds(off, 16)] = one16

        def zunit(t, carry):
            u = sid + t * _NS

            @pl.when(u < nu)
            def _():
                r0 = u * _CB
                pltpu.sync_copy(blk_v, acc1.at[pl.ds(r0, _CB)])
                pltpu.sync_copy(blk_v, acc2.at[pl.ds(r0, _CB)])
                pltpu.sync_copy(cnt_v, hist_c.at[pl.ds(r0, _CB)])
                pltpu.sync_copy(cnt_v, hist_r.at[pl.ds(r0, _CB)])
            return carry

        lax.fori_loop(0, nt, zunit, 0)

        pltpu.sync_copy(ridx_hbm.at[sid], rowv)
        pltpu.sync_copy(cidx_hbm.at[sid], colv)
        plsc.subcore_barrier()

        def g1(c, buf):
            @pl.when(cid == 0)
            def _():
                pltpu.make_async_copy(
                    ta_hbm.at[rowv.at[c]], rows_v.at[buf], sem).start()

            @pl.when(cid == 1)
            def _():
                pltpu.make_async_copy(
                    tb_hbm.at[rowv.at[c]], rows_v.at[buf], sem).start()

        g1(0, 0)

        def chunk1(c, carry):
            buf = lax.rem(c, 2)

            @pl.when(c + 1 < nch)
            def _():
                g1(c + 1, 1 - buf)

            pltpu.make_async_copy(
                ta_hbm.at[rowv.at[c]], rows_v.at[buf], sem).wait()
            pltpu.sync_copy(rows_v.at[buf], acc1.at[colv.at[c]], add=True)
            pltpu.sync_copy(ones_v, hist_c.at[colv.at[c]], add=True)
            pltpu.sync_copy(ones_v, hist_r.at[rowv.at[c]], add=True)
            return carry

        lax.fori_loop(0, nch, chunk1, 0)
        plsc.subcore_barrier()

        def cunit(t, carry):
            u = sid + t * _NS

            @pl.when(u < nu)
            def _():
                r0 = u * _CB
                pltpu.sync_copy(acc1.at[pl.ds(r0, _CB)], blk_v)
                pltpu.sync_copy(hist_c.at[pl.ds(r0, _CB)], cnt_v)
                for g in range(_CB // 16):
                    cnt = cnt_v[pl.ds(g * 16, 16)]
                    inv = jnp.where(cnt == 0, 0.0,
                                    1.0 / jnp.where(cnt == 0, 1.0, cnt))
                    for j in range(16):
                        r = g * 16 + j
                        s = inv[j]
                        for seg in range(_HW // 16):
                            v = blk_v[r, pl.ds(seg * 16, 16)]
                            blk_v[r, pl.ds(seg * 16, 16)] = v * s
                pltpu.sync_copy(blk_v, acc1.at[pl.ds(r0, _CB)])
            return carry

        lax.fori_loop(0, nt, cunit, 0)
        plsc.subcore_barrier()

        def g2(c, buf):
            pltpu.make_async_copy(
                acc1.at[colv.at[c]], rows_v.at[buf], sem).start()

        g2(0, 0)

        def chunk2(c, carry):
            buf = lax.rem(c, 2)

            @pl.when(c + 1 < nch)
            def _():
                g2(c + 1, 1 - buf)

            pltpu.make_async_copy(
                acc1.at[colv.at[c]], rows_v.at[buf], sem).wait()
            pltpu.sync_copy(rows_v.at[buf], acc2.at[rowv.at[c]], add=True)
            return carry

        lax.fori_loop(0, nch, chunk2, 0)
        plsc.subcore_barrier()

        base = sid * per_sub

        @pl.when(cid == 0)
        def _():
            pltpu.sync_copy(acc2.at[pl.ds(base, per_sub)],
                            qa_hbm.at[pl.ds(base, per_sub)])

        @pl.when(cid == 1)
        def _():
            pltpu.sync_copy(acc2.at[pl.ds(base, per_sub)],
                            qb_hbm.at[pl.ds(base, per_sub)])

        @pl.when(jnp.logical_and(cid == 0, sid == 0))
        def _():
            pltpu.sync_copy(hist_r, dv_hbm)

    return body(ta, tb, ridx, cidx)


def _back_body(src_ref, h_ref, qa_ref, qb_ref, dv_ref, bc_ref, g2_ref,
               be2_ref, w2t_ref, b2_ref, lw_ref, lb_ref, o_ref):
    dv = dv_ref[...]
    dinv = jnp.where(dv == 0, 0.0, 1.0 / jnp.where(dv == 0, 1.0, dv))
    outv = jnp.concatenate([qa_ref[...], qb_ref[...]], axis=1) * dinv
    hh = h_ref[...] + outv + bc_ref[...]
    hh = hh * (_S1 * g2_ref[...]) + be2_ref[...]
    g = jnp.dot(hh, w2t_ref[...], preferred_element_type=jnp.float32)
    g = _leaky(g + b2_ref[...])
    o = src_ref[...] + g
    mu = jnp.mean(o, axis=1, keepdims=True)
    var = jnp.mean((o - mu) ** 2, axis=1, keepdims=True)
    o_ref[...] = (o - mu) / jnp.sqrt(var + EPS) * lw_ref[...] + lb_ref[...]


def _tc_back(src2d, h, qa, qb, dv2d, bc, g2, be2, w2t, b2, lw, lb, n, blk):
    vec = pl.BlockSpec((1, 128), lambda i: (0, 0))
    return pl.pallas_call(
        _back_body,
        grid=(n // blk,),
        in_specs=[
            pl.BlockSpec((blk, 128), lambda i: (i, 0)),
            pl.BlockSpec((blk, 128), lambda i: (i, 0)),
            pl.BlockSpec((blk, _HW), lambda i: (i, 0)),
            pl.BlockSpec((blk, _HW), lambda i: (i, 0)),
            pl.BlockSpec((blk, 1), lambda i: (i, 0)),
            vec, vec, vec,
            pl.BlockSpec((128, 128), lambda i: (0, 0)),
            vec, vec, vec,
        ],
        out_specs=pl.BlockSpec((blk, 128), lambda i: (i, 0)),
        out_shape=jax.ShapeDtypeStruct((n, 128), jnp.float32),
    )(src2d, h, qa, qb, dv2d, bc, g2, be2, w2t, b2, lw, lb)


def kernel(x, hyperedge_all, lin1_W, lin1_b, bn1_w, bn1_b, hconv_W, hconv_b,
           bn2_w, bn2_b, lin2_W, lin2_b, ln_w, ln_b):
    b_, n, c = x.shape
    nnz = hyperedge_all.shape[1]
    per_sub_e = nnz // _NS
    k = 80
    nch = per_sub_e // k

    x2d = x.reshape(n, b_ * c)
    ridx = hyperedge_all[0].reshape(_NS, nch, k)
    cidx = hyperedge_all[1].reshape(_NS, nch, k)

    blk = 1000
    h, ta, tb = _tc_front(
        x2d, lin1_W.T, lin1_b.reshape(1, -1), bn1_w.reshape(1, -1),
        bn1_b.reshape(1, -1), hconv_W.T, n, blk)

    qa, qb, dv = _sc_mega(ta, tb, ridx, cidx)

    out2d = _tc_back(
        x2d, h, qa, qb, dv.reshape(n, 1), hconv_b.reshape(1, -1),
        bn2_w.reshape(1, -1), bn2_b.reshape(1, -1), lin2_W.T,
        lin2_b.reshape(1, -1), ln_w.reshape(1, -1), ln_b.reshape(1, -1),
        n, blk)
    return out2d.reshape(b_, n, c)
